# baseline (speedup 1.0000x reference)
import jax
import jax.numpy as jnp
from jax import lax
from jax.experimental import pallas as pl
from jax.experimental.pallas import tpu as pltpu
from jax.experimental.pallas import tpu_sc as plsc

_B = 16384
_D = 64

_info = plsc.get_sparse_core_info()
_NC, _NS = _info.num_cores, _info.num_subcores
_NW = _NC * _NS
_BPW = _B // _NW


def _gather3_body(u_idx, p_idx, n_idx, users_tbl, items_tbl,
                  out_u, out_p, out_n,
                  idx_u, idx_p, idx_n, rows_u, rows_p, rows_n,
                  sem_u, sem_p, sem_n):
    wid = lax.axis_index("s") * _NC + lax.axis_index("c")
    base = wid * _BPW
    sl = pl.ds(base, _BPW)
    pltpu.sync_copy(u_idx.at[sl], idx_u)
    pltpu.sync_copy(p_idx.at[sl], idx_p)
    pltpu.sync_copy(n_idx.at[sl], idx_n)
    cu = pltpu.async_copy(users_tbl.at[idx_u], rows_u, sem_u)
    cp = pltpu.async_copy(items_tbl.at[idx_p], rows_p, sem_p)
    cn = pltpu.async_copy(items_tbl.at[idx_n], rows_n, sem_n)
    cu.wait()
    pltpu.sync_copy(rows_u, out_u.at[sl])
    cp.wait()
    pltpu.sync_copy(rows_p, out_p.at[sl])
    cn.wait()
    pltpu.sync_copy(rows_n, out_n.at[sl])


_out_struct = jax.ShapeDtypeStruct((_B, _D), jnp.float32)

_gather3 = pl.kernel(
    _gather3_body,
    mesh=plsc.VectorSubcoreMesh(core_axis_name="c", subcore_axis_name="s"),
    compiler_params=pltpu.CompilerParams(use_tc_tiling_on_sc=False),
    out_type=(_out_struct, _out_struct, _out_struct),
    scratch_types=[
        pltpu.VMEM((_BPW,), jnp.int32),
        pltpu.VMEM((_BPW,), jnp.int32),
        pltpu.VMEM((_BPW,), jnp.int32),
        pltpu.VMEM((_BPW, _D), jnp.float32),
        pltpu.VMEM((_BPW, _D), jnp.float32),
        pltpu.VMEM((_BPW, _D), jnp.float32),
        pltpu.SemaphoreType.DMA,
        pltpu.SemaphoreType.DMA,
        pltpu.SemaphoreType.DMA,
    ],
)


def kernel(batch_users, batch_pos_items, batch_neg_items, users_table, items_table):
    u = batch_users.astype(jnp.int32)
    p = batch_pos_items.astype(jnp.int32)
    n = batch_neg_items.astype(jnp.int32)
    return _gather3(u, p, n, users_table, items_table)
